# baseline (speedup 1.0000x reference)
import functools

import jax
import jax.numpy as jnp
from jax import lax
from jax.experimental import pallas as pl
from jax.experimental.pallas import tpu as pltpu
from jax.experimental.pallas import tpu_sc as plsc

_NUM_CORES = 2
_NUM_SUBCORES = 16
_NUM_WORKERS = _NUM_CORES * _NUM_SUBCORES
_LANES = 16


def kernel(user_indices, item_indices, user_table, item_table):
    batch = user_indices.shape[0]
    vocab, embed = user_table.shape
    b_per_w = batch // _NUM_WORKERS

    user_indices = user_indices.astype(jnp.int32)
    item_indices = item_indices.astype(jnp.int32)
    utf = user_table.T.reshape(-1)
    itf = item_table.T.reshape(-1)

    mesh = plsc.VectorSubcoreMesh(core_axis_name="c", subcore_axis_name="s")

    @functools.partial(
        pl.kernel,
        mesh=mesh,
        out_type=jax.ShapeDtypeStruct((embed, batch), jnp.float32),
        scratch_types=[
            pltpu.VMEM((b_per_w,), jnp.int32),
            pltpu.VMEM((b_per_w,), jnp.int32),
            pltpu.VMEM((embed * b_per_w,), jnp.int32),
            pltpu.VMEM((embed * b_per_w,), jnp.int32),
            pltpu.VMEM((embed * b_per_w,), jnp.float32),
            pltpu.VMEM((embed * b_per_w,), jnp.float32),
            pltpu.SemaphoreType.DMA,
            pltpu.SemaphoreType.DMA,
            pltpu.SemaphoreType.DMA,
        ],
    )
    def gmf_kernel(uidx_hbm, iidx_hbm, utab_hbm, itab_hbm, out_hbm,
                   uidx_v, iidx_v, uoff_v, ioff_v, ubuf_v, ibuf_v,
                   sem_u, sem_i, sem_o):
        wid = lax.axis_index("s") * _NUM_CORES + lax.axis_index("c")
        base = wid * b_per_w

        pltpu.sync_copy(uidx_hbm.at[pl.ds(base, b_per_w)], uidx_v)
        pltpu.sync_copy(iidx_hbm.at[pl.ds(base, b_per_w)], iidx_v)

        for j in range(embed):
            @pl.loop(0, b_per_w, step=_LANES)
            def _(c, j=j):
                src = pl.ds(c, _LANES)
                dst = pl.ds(j * b_per_w + c, _LANES)
                uoff_v[dst] = uidx_v[src] + j * vocab
                ioff_v[dst] = iidx_v[src] + j * vocab

        cu = pltpu.async_copy(utab_hbm.at[uoff_v], ubuf_v, sem_u)
        ci = pltpu.async_copy(itab_hbm.at[ioff_v], ibuf_v, sem_i)
        cu.wait()
        ci.wait()

        @pl.loop(0, embed * b_per_w, step=_LANES)
        def _(c):
            sl = pl.ds(c, _LANES)
            ubuf_v[sl] = ubuf_v[sl] * ibuf_v[sl]

        ocopies = []
        for j in range(embed):
            ocopies.append(
                pltpu.async_copy(ubuf_v.at[pl.ds(j * b_per_w, b_per_w)],
                                 out_hbm.at[j, pl.ds(base, b_per_w)], sem_o))
        for c in ocopies:
            c.wait()

    out_t = gmf_kernel(user_indices, item_indices, utf, itf)
    return out_t.T
